# baseline (speedup 1.0000x reference)
import functools

import jax
import jax.numpy as jnp
from jax import lax
from jax.experimental import pallas as pl
from jax.experimental.pallas import tpu as pltpu
from jax.experimental.pallas import tpu_sc as plsc

NUM_BINS = 64
GROUP = 16
HIDDEN = 128

B, C, H, W = 2, 96, 512, 512
BC = B * C
NPIX = H * W
BLK = 16

_NC, _NS, _L = 2, 16, 16
_NW = _NC * _NS
CPW = BC // _NW
CHUNK = 32768
NCHUNK = NPIX // CHUNK


def _stats_body(x_ref, mn_ref, mx_ref, bs_ref):
    xb = x_ref[0]
    mn_ref[0] = jnp.min(xb.reshape(BLK * 4, 8, W), axis=0)
    mx_ref[0] = jnp.max(xb.reshape(BLK * 4, 8, W), axis=0)
    rs = xb.reshape(H // BLK, BLK, W).sum(axis=1)
    wi = lax.broadcasted_iota(jnp.int32, (W, W // BLK), 0)
    ci = lax.broadcasted_iota(jnp.int32, (W, W // BLK), 1)
    P = (wi // BLK == ci).astype(jnp.float32)
    bs_ref[0] = jnp.dot(rs, P, preferred_element_type=jnp.float32)


def _run_stats(xf):
    return pl.pallas_call(
        _stats_body,
        grid=(BC,),
        in_specs=[pl.BlockSpec((1, H, W), lambda i: (i, 0, 0))],
        out_specs=[
            pl.BlockSpec((1, 8, W), lambda i: (i, 0, 0)),
            pl.BlockSpec((1, 8, W), lambda i: (i, 0, 0)),
            pl.BlockSpec((1, H // BLK, W // BLK), lambda i: (i, 0, 0)),
        ],
        out_shape=[
            jax.ShapeDtypeStruct((BC, 8, W), jnp.float32),
            jax.ShapeDtypeStruct((BC, 8, W), jnp.float32),
            jax.ShapeDtypeStruct((BC, H // BLK, W // BLK), jnp.float32),
        ],
        compiler_params=pltpu.CompilerParams(
            dimension_semantics=("arbitrary",)),
    )(xf)


def _lut_body(bs_ref, mn_ref, mx_ref, w1_ref, b1_ref, w2_ref, b2_ref,
              w3_ref, b3_ref, alpha_ref, lut_ref, lutd_ref, s_ref, t_ref):
    K = NUM_BINS
    G = GROUP
    xmn = jnp.min(mn_ref[...], axis=1, keepdims=True)
    xmx = jnp.max(mx_ref[...], axis=1, keepdims=True)
    rng = xmx - xmn
    inv = 1.0 / (rng + 1e-6)
    xs = (bs_ref[...] * (1.0 / (BLK * BLK)) - xmn) * inv
    ji = lax.broadcasted_iota(jnp.int32, (B * G, BC), 0)
    bci = lax.broadcasted_iota(jnp.int32, (B * G, BC), 1)
    bg = (bci // C) * G + (bci % C) // (C // G)
    gsel = jnp.where(bg == ji, 1.0 / (C // G), 0.0)
    xg = jnp.dot(gsel, xs, preferred_element_type=jnp.float32)
    idx = jnp.clip(jnp.round(xg * (K - 1)).astype(jnp.int32), 0, K - 1)
    ki = lax.broadcasted_iota(jnp.int32, (B * G, xg.shape[1], K), 2)
    oh = (idx[:, :, None] == ki).astype(jnp.float32)
    hist = oh.sum(axis=1)
    pdf = hist / (hist.sum(axis=-1, keepdims=True) + 1e-6)
    r0 = lax.broadcasted_iota(jnp.int32, (K, K), 0)
    r1 = lax.broadcasted_iota(jnp.int32, (K, K), 1)
    T = (r0 <= r1).astype(jnp.float32)
    cdf = jnp.dot(pdf, T, preferred_element_type=jnp.float32)
    dc = 0.5 * (cdf[:G] + cdf[G:])
    z2 = jnp.zeros((G, 2), jnp.float32)
    dpad = jnp.concatenate([z2, dc, z2], axis=1)
    col = jnp.concatenate([dpad[:, t:t + K] for t in range(5)], axis=0)
    h = jnp.maximum(
        jnp.dot(w1_ref[...], col, preferred_element_type=jnp.float32)
        + b1_ref[...], 0.0)
    z2h = jnp.zeros((HIDDEN, 2), jnp.float32)
    hpad = jnp.concatenate([z2h, h, z2h], axis=1)
    w2 = w2_ref[...]
    h2 = b2_ref[...]
    for t in range(5):
        h2 = h2 + w2[:, t:t + 1] * hpad[:, t:t + K]
    h2 = jnp.maximum(h2, 0.0)
    delta = (jnp.dot(w3_ref[...], h2, preferred_element_type=jnp.float32)
             + b3_ref[...])
    sp = jnp.maximum(delta, 0.0) + jnp.log(1.0 + jnp.exp(-jnp.abs(delta)))
    cdf2 = jnp.dot(sp, T, preferred_element_type=jnp.float32)
    cdf2 = cdf2 / (cdf2[:, K - 1:K] + 1e-6)
    ident = lax.broadcasted_iota(jnp.int32, (G, K), 1).astype(jnp.float32)
    ident = ident * (1.0 / (K - 1))
    a = 1.0 / (1.0 + jnp.exp(-jnp.full((G, K), alpha_ref[0, 0])))
    lut2 = a * (cdf2 + ident) + (1.0 - a) * ident
    lutc = jnp.broadcast_to(lut2[:, None, :], (G, C // G, K)).reshape(C, K)
    lutbc = jnp.broadcast_to(lutc[None], (B, C, K)).reshape(BC, K)
    lut3 = lutbc * rng + xmn
    lut_ref[...] = lut3
    lutd_ref[...] = jnp.concatenate(
        [lut3[:, 1:] - lut3[:, :-1], jnp.zeros((BC, 1), jnp.float32)], axis=1)
    s = (K - 1.0) * inv
    s_ref[...] = jnp.broadcast_to(s, (BC, _L))
    t_ref[...] = jnp.broadcast_to(-(xmn * s), (BC, _L))


def _run_lut(bs2, xmn, xmx, w1e, b1c, w2e, b2c, w3r, b3c, alpha2):
    n_in = 9
    return pl.pallas_call(
        _lut_body,
        in_specs=[pl.BlockSpec(memory_space=pltpu.VMEM)] * n_in
        + [pl.BlockSpec(memory_space=pltpu.SMEM)],
        out_specs=[pl.BlockSpec(memory_space=pltpu.VMEM)] * 4,
        out_shape=[
            jax.ShapeDtypeStruct((BC, NUM_BINS), jnp.float32),
            jax.ShapeDtypeStruct((BC, NUM_BINS), jnp.float32),
            jax.ShapeDtypeStruct((BC, _L), jnp.float32),
            jax.ShapeDtypeStruct((BC, _L), jnp.float32),
        ],
    )(bs2, xmn, xmx, w1e, b1c, w2e, b2c, w3r, b3c, alpha2)


def _pix_body(x_hbm, lut_hbm, lutd_hbm, s_hbm, t_hbm, out_hbm,
              lut_v, lutd_v, s_v, t_v, b0, b1, b2,
              si0, si1, si2, so0, so1, so2):
    wid = lax.axis_index("s") * _NC + lax.axis_index("c")
    cbase = wid * CPW
    pltpu.sync_copy(lut_hbm.at[pl.ds(cbase * NUM_BINS, CPW * NUM_BINS)], lut_v)
    pltpu.sync_copy(lutd_hbm.at[pl.ds(cbase * NUM_BINS, CPW * NUM_BINS)],
                    lutd_v)
    pltpu.sync_copy(s_hbm.at[pl.ds(cbase * _L, CPW * _L)], s_v)
    pltpu.sync_copy(t_hbm.at[pl.ds(cbase * _L, CPW * _L)], t_v)
    nch = CPW * NCHUNK
    bufs, sins, souts = (b0, b1, b2), (si0, si1, si2), (so0, so1, so2)

    def in_sl(ch):
        return x_hbm.at[cbase + ch // NCHUNK, ch % NCHUNK, :, :]

    def out_sl(ch):
        return out_hbm.at[cbase + ch // NCHUNK, ch % NCHUNK, :, :]

    pltpu.async_copy(in_sl(0), b0, si0)
    pltpu.async_copy(in_sl(1), b1, si1)

    def group(g, carry):
        for b in range(3):
            ch = 3 * g + b
            buf, si, so = bufs[b], sins[b], souts[b]
            nb = (b + 2) % 3
            pltpu.make_async_copy(in_sl(ch), buf, si).wait()
            cl = ch // NCHUNK
            sv = s_v[pl.ds(cl * _L, _L)]
            tv = t_v[pl.ds(cl * _L, _L)]
            base_vec = jnp.full((_L,), cl * NUM_BINS, jnp.int32)

            @plsc.parallel_loop(0, CHUNK, _L, unroll=8)
            def pix(off, buf=buf, sv=sv, tv=tv, base_vec=base_vec):
                r = off // W
                c = off % W
                xv = buf[r, pl.ds(c, _L)]
                pos = xv * sv + tv
                idl = pos.astype(jnp.int32)
                frac = pos - idl.astype(jnp.float32)
                fl = base_vec + idl
                lo = plsc.load_gather(lut_v, [fl])
                dd = plsc.load_gather(lutd_v, [fl])
                buf[r, pl.ds(c, _L)] = lo + frac * dd

            pltpu.async_copy(buf, out_sl(ch), so)

            @pl.when(ch + 2 < nch)
            def _():
                @pl.when(ch >= 1)
                def _():
                    pltpu.make_async_copy(
                        bufs[nb], out_sl(ch - 1), souts[nb]).wait()
                pltpu.async_copy(in_sl(ch + 2), bufs[nb], sins[nb])
        return carry

    lax.fori_loop(0, nch // 3, group, 0)
    for j in range(3):
        pltpu.make_async_copy(bufs[j], out_sl(nch - 3 + j), souts[j]).wait()


_CROWS = CHUNK // W

_pix_kernel = functools.partial(
    pl.kernel,
    out_type=jax.ShapeDtypeStruct((BC, NCHUNK, _CROWS, W), jnp.float32),
    mesh=plsc.VectorSubcoreMesh(
        core_axis_name="c", subcore_axis_name="s",
        num_cores=_NC, num_subcores=_NS),
    compiler_params=pltpu.CompilerParams(
        needs_layout_passes=False, use_tc_tiling_on_sc=True),
    scratch_types=[
        pltpu.VMEM((CPW * NUM_BINS,), jnp.float32),
        pltpu.VMEM((CPW * NUM_BINS,), jnp.float32),
        pltpu.VMEM((CPW * _L,), jnp.float32),
        pltpu.VMEM((CPW * _L,), jnp.float32),
        pltpu.VMEM((_CROWS, W), jnp.float32),
        pltpu.VMEM((_CROWS, W), jnp.float32),
        pltpu.VMEM((_CROWS, W), jnp.float32),
        pltpu.SemaphoreType.DMA,
        pltpu.SemaphoreType.DMA,
        pltpu.SemaphoreType.DMA,
        pltpu.SemaphoreType.DMA,
        pltpu.SemaphoreType.DMA,
        pltpu.SemaphoreType.DMA,
    ],
)(_pix_body)


@jax.jit
def kernel(x, W1, b1, W2, b2, W3, b3, alpha):
    xf = x.reshape(BC, H, W)
    mn, mx, bs = _run_stats(xf)
    xmn = mn.reshape(BC, 8 * W)
    xmx = mx.reshape(BC, 8 * W)
    bs2 = bs.reshape(BC, (H // BLK) * (W // BLK))
    w1e = W1[:, :, 2, :].transpose(0, 2, 1).reshape(HIDDEN, 5 * GROUP)
    w2e = W2[:, 0, 2, :]
    w3r = W3[:, :, 0, 0]
    lut3, lutd, s_rep, t_rep = _run_lut(
        bs2, xmn, xmx, w1e, b1.reshape(HIDDEN, 1), w2e,
        b2.reshape(HIDDEN, 1), w3r, b3.reshape(GROUP, 1),
        alpha.reshape(1, 1))
    out = _pix_kernel(x.reshape(BC, NCHUNK, H // NCHUNK, W), lut3.reshape(-1),
                      lutd.reshape(-1), s_rep.reshape(-1), t_rep.reshape(-1))
    return out.reshape(B, C, H, W)


# speedup vs baseline: 1.1432x; 1.1432x over previous
import functools

import jax
import jax.numpy as jnp
from jax import lax
from jax.experimental import pallas as pl
from jax.experimental.pallas import tpu as pltpu
from jax.experimental.pallas import tpu_sc as plsc

NUM_BINS = 64
GROUP = 16
HIDDEN = 128

B, C, H, W = 2, 96, 512, 512
BC = B * C
NPIX = H * W
BLK = 16

_NC, _NS, _L = 2, 16, 16
_NW = _NC * _NS
CPW = BC // _NW
CHUNK = 32768
NCHUNK = NPIX // CHUNK


_IPS = 2


def _stats_body(x_ref, mn_ref, mx_ref, bs_ref):
    xb = x_ref[...]
    mn_ref[0] = jnp.min(xb.reshape(_IPS, BLK * 4, 8, W),
                        axis=1).reshape(_IPS * 8, W)
    mx_ref[0] = jnp.max(xb.reshape(_IPS, BLK * 4, 8, W),
                        axis=1).reshape(_IPS * 8, W)
    rs = xb.reshape(_IPS * (H // BLK), BLK, W).sum(axis=1)
    wi = lax.broadcasted_iota(jnp.int32, (W, W // BLK), 0)
    ci = lax.broadcasted_iota(jnp.int32, (W, W // BLK), 1)
    P = (wi // BLK == ci).astype(jnp.float32)
    bs_ref[0] = jnp.dot(rs, P, preferred_element_type=jnp.float32)


def _run_stats(xf):
    ng = BC // _IPS
    return pl.pallas_call(
        _stats_body,
        grid=(ng,),
        in_specs=[pl.BlockSpec((_IPS, H, W), lambda i: (i, 0, 0))],
        out_specs=[
            pl.BlockSpec((1, _IPS * 8, W), lambda i: (i, 0, 0)),
            pl.BlockSpec((1, _IPS * 8, W), lambda i: (i, 0, 0)),
            pl.BlockSpec((1, _IPS * (H // BLK), W // BLK),
                         lambda i: (i, 0, 0)),
        ],
        out_shape=[
            jax.ShapeDtypeStruct((ng, _IPS * 8, W), jnp.float32),
            jax.ShapeDtypeStruct((ng, _IPS * 8, W), jnp.float32),
            jax.ShapeDtypeStruct((ng, _IPS * (H // BLK), W // BLK),
                                 jnp.float32),
        ],
        compiler_params=pltpu.CompilerParams(
            dimension_semantics=("arbitrary",)),
    )(xf)


def _lut_body(bs_ref, mn_ref, mx_ref, w1_ref, b1_ref, w2_ref, b2_ref,
              w3_ref, b3_ref, alpha_ref, lut_ref, lutd_ref, s_ref, t_ref):
    K = NUM_BINS
    G = GROUP
    xmn = jnp.min(mn_ref[...], axis=1, keepdims=True)
    xmx = jnp.max(mx_ref[...], axis=1, keepdims=True)
    rng = xmx - xmn
    inv = 1.0 / (rng + 1e-6)
    xs = (bs_ref[...] * (1.0 / (BLK * BLK)) - xmn) * inv
    ji = lax.broadcasted_iota(jnp.int32, (B * G, BC), 0)
    bci = lax.broadcasted_iota(jnp.int32, (B * G, BC), 1)
    bg = (bci // C) * G + (bci % C) // (C // G)
    gsel = jnp.where(bg == ji, 1.0 / (C // G), 0.0)
    xg = jnp.dot(gsel, xs, preferred_element_type=jnp.float32)
    idx = jnp.clip(jnp.round(xg * (K - 1)).astype(jnp.int32), 0, K - 1)
    ki = lax.broadcasted_iota(jnp.int32, (B * G, xg.shape[1], K), 2)
    oh = (idx[:, :, None] == ki).astype(jnp.float32)
    hist = oh.sum(axis=1)
    pdf = hist / (hist.sum(axis=-1, keepdims=True) + 1e-6)
    r0 = lax.broadcasted_iota(jnp.int32, (K, K), 0)
    r1 = lax.broadcasted_iota(jnp.int32, (K, K), 1)
    T = (r0 <= r1).astype(jnp.float32)
    cdf = jnp.dot(pdf, T, preferred_element_type=jnp.float32)
    dc = 0.5 * (cdf[:G] + cdf[G:])
    z2 = jnp.zeros((G, 2), jnp.float32)
    dpad = jnp.concatenate([z2, dc, z2], axis=1)
    col = jnp.concatenate([dpad[:, t:t + K] for t in range(5)], axis=0)
    h = jnp.maximum(
        jnp.dot(w1_ref[...], col, preferred_element_type=jnp.float32)
        + b1_ref[...], 0.0)
    z2h = jnp.zeros((HIDDEN, 2), jnp.float32)
    hpad = jnp.concatenate([z2h, h, z2h], axis=1)
    w2 = w2_ref[...]
    h2 = b2_ref[...]
    for t in range(5):
        h2 = h2 + w2[:, t:t + 1] * hpad[:, t:t + K]
    h2 = jnp.maximum(h2, 0.0)
    delta = (jnp.dot(w3_ref[...], h2, preferred_element_type=jnp.float32)
             + b3_ref[...])
    sp = jnp.maximum(delta, 0.0) + jnp.log(1.0 + jnp.exp(-jnp.abs(delta)))
    cdf2 = jnp.dot(sp, T, preferred_element_type=jnp.float32)
    cdf2 = cdf2 / (cdf2[:, K - 1:K] + 1e-6)
    ident = lax.broadcasted_iota(jnp.int32, (G, K), 1).astype(jnp.float32)
    ident = ident * (1.0 / (K - 1))
    a = 1.0 / (1.0 + jnp.exp(-jnp.full((G, K), alpha_ref[0, 0])))
    lut2 = a * (cdf2 + ident) + (1.0 - a) * ident
    lutc = jnp.broadcast_to(lut2[:, None, :], (G, C // G, K)).reshape(C, K)
    lutbc = jnp.broadcast_to(lutc[None], (B, C, K)).reshape(BC, K)
    lut3 = lutbc * rng + xmn
    lut_ref[...] = lut3
    lutd_ref[...] = jnp.concatenate(
        [lut3[:, 1:] - lut3[:, :-1], jnp.zeros((BC, 1), jnp.float32)], axis=1)
    s = (K - 1.0) * inv
    s_ref[...] = jnp.broadcast_to(s, (BC, _L))
    t_ref[...] = jnp.broadcast_to(-(xmn * s), (BC, _L))


def _run_lut(bs2, xmn, xmx, w1e, b1c, w2e, b2c, w3r, b3c, alpha2):
    n_in = 9
    return pl.pallas_call(
        _lut_body,
        in_specs=[pl.BlockSpec(memory_space=pltpu.VMEM)] * n_in
        + [pl.BlockSpec(memory_space=pltpu.SMEM)],
        out_specs=[pl.BlockSpec(memory_space=pltpu.VMEM)] * 4,
        out_shape=[
            jax.ShapeDtypeStruct((BC, NUM_BINS), jnp.float32),
            jax.ShapeDtypeStruct((BC, NUM_BINS), jnp.float32),
            jax.ShapeDtypeStruct((BC, _L), jnp.float32),
            jax.ShapeDtypeStruct((BC, _L), jnp.float32),
        ],
    )(bs2, xmn, xmx, w1e, b1c, w2e, b2c, w3r, b3c, alpha2)


def _pix_body(x_hbm, lut_hbm, lutd_hbm, s_hbm, t_hbm, out_hbm,
              lut_v, lutd_v, s_v, t_v, b0, b1, b2,
              si0, si1, si2, so0, so1, so2):
    wid = lax.axis_index("s") * _NC + lax.axis_index("c")
    cbase = wid * CPW
    pltpu.sync_copy(lut_hbm.at[pl.ds(cbase * NUM_BINS, CPW * NUM_BINS)], lut_v)
    pltpu.sync_copy(lutd_hbm.at[pl.ds(cbase * NUM_BINS, CPW * NUM_BINS)],
                    lutd_v)
    pltpu.sync_copy(s_hbm.at[pl.ds(cbase * _L, CPW * _L)], s_v)
    pltpu.sync_copy(t_hbm.at[pl.ds(cbase * _L, CPW * _L)], t_v)
    nch = CPW * NCHUNK
    bufs, sins, souts = (b0, b1, b2), (si0, si1, si2), (so0, so1, so2)

    def in_sl(ch):
        return x_hbm.at[cbase + ch // NCHUNK, ch % NCHUNK, :, :]

    def out_sl(ch):
        return out_hbm.at[cbase + ch // NCHUNK, ch % NCHUNK, :, :]

    pltpu.async_copy(in_sl(0), b0, si0)
    pltpu.async_copy(in_sl(1), b1, si1)

    def group(g, carry):
        for b in range(3):
            ch = 3 * g + b
            buf, si, so = bufs[b], sins[b], souts[b]
            nb = (b + 2) % 3
            pltpu.make_async_copy(in_sl(ch), buf, si).wait()
            cl = ch // NCHUNK
            sv = s_v[pl.ds(cl * _L, _L)]
            tv = t_v[pl.ds(cl * _L, _L)]
            base_vec = jnp.full((_L,), cl * NUM_BINS, jnp.int32)

            @plsc.parallel_loop(0, CHUNK, _L, unroll=8)
            def pix(off, buf=buf, sv=sv, tv=tv, base_vec=base_vec):
                r = off // W
                c = off % W
                xv = buf[r, pl.ds(c, _L)]
                pos = xv * sv + tv
                idl = pos.astype(jnp.int32)
                frac = pos - idl.astype(jnp.float32)
                fl = base_vec + idl
                lo = plsc.load_gather(lut_v, [fl])
                dd = plsc.load_gather(lutd_v, [fl])
                buf[r, pl.ds(c, _L)] = lo + frac * dd

            pltpu.async_copy(buf, out_sl(ch), so)

            @pl.when(ch + 2 < nch)
            def _():
                @pl.when(ch >= 1)
                def _():
                    pltpu.make_async_copy(
                        bufs[nb], out_sl(ch - 1), souts[nb]).wait()
                pltpu.async_copy(in_sl(ch + 2), bufs[nb], sins[nb])
        return carry

    lax.fori_loop(0, nch // 3, group, 0)
    for j in range(3):
        pltpu.make_async_copy(bufs[j], out_sl(nch - 3 + j), souts[j]).wait()


_CROWS = CHUNK // W

_pix_kernel = functools.partial(
    pl.kernel,
    out_type=jax.ShapeDtypeStruct((BC, NCHUNK, _CROWS, W), jnp.float32),
    mesh=plsc.VectorSubcoreMesh(
        core_axis_name="c", subcore_axis_name="s",
        num_cores=_NC, num_subcores=_NS),
    compiler_params=pltpu.CompilerParams(
        needs_layout_passes=False, use_tc_tiling_on_sc=True),
    scratch_types=[
        pltpu.VMEM((CPW * NUM_BINS,), jnp.float32),
        pltpu.VMEM((CPW * NUM_BINS,), jnp.float32),
        pltpu.VMEM((CPW * _L,), jnp.float32),
        pltpu.VMEM((CPW * _L,), jnp.float32),
        pltpu.VMEM((_CROWS, W), jnp.float32),
        pltpu.VMEM((_CROWS, W), jnp.float32),
        pltpu.VMEM((_CROWS, W), jnp.float32),
        pltpu.SemaphoreType.DMA,
        pltpu.SemaphoreType.DMA,
        pltpu.SemaphoreType.DMA,
        pltpu.SemaphoreType.DMA,
        pltpu.SemaphoreType.DMA,
        pltpu.SemaphoreType.DMA,
    ],
)(_pix_body)


@jax.jit
def kernel(x, W1, b1, W2, b2, W3, b3, alpha):
    xf = x.reshape(BC, H, W)
    mn, mx, bs = _run_stats(xf)
    xmn = mn.reshape(BC, 8 * W)
    xmx = mx.reshape(BC, 8 * W)
    bs2 = bs.reshape(BC, (H // BLK) * (W // BLK))
    w1e = W1[:, :, 2, :].transpose(0, 2, 1).reshape(HIDDEN, 5 * GROUP)
    w2e = W2[:, 0, 2, :]
    w3r = W3[:, :, 0, 0]
    lut3, lutd, s_rep, t_rep = _run_lut(
        bs2, xmn, xmx, w1e, b1.reshape(HIDDEN, 1), w2e,
        b2.reshape(HIDDEN, 1), w3r, b3.reshape(GROUP, 1),
        alpha.reshape(1, 1))
    out = _pix_kernel(x.reshape(BC, NCHUNK, H // NCHUNK, W), lut3.reshape(-1),
                      lutd.reshape(-1), s_rep.reshape(-1), t_rep.reshape(-1))
    return out.reshape(B, C, H, W)


# speedup vs baseline: 1.2455x; 1.0895x over previous
import functools

import jax
import jax.numpy as jnp
from jax import lax
from jax.experimental import pallas as pl
from jax.experimental.pallas import tpu as pltpu
from jax.experimental.pallas import tpu_sc as plsc

NUM_BINS = 64
GROUP = 16
HIDDEN = 128

B, C, H, W = 2, 96, 512, 512
BC = B * C
NPIX = H * W
BLK = 16

_NC, _NS, _L = 2, 16, 16
_NW = _NC * _NS
CPW = BC // _NW
CHUNK = 32768
NCHUNK = NPIX // CHUNK


_IPS = 4


def _stats_body(x_ref, mn_ref, mx_ref, bs_ref):
    xb = x_ref[...]
    mn_ref[0] = jnp.min(xb.reshape(_IPS, BLK * 4, 8, W),
                        axis=1).reshape(_IPS * 8, W)
    mx_ref[0] = jnp.max(xb.reshape(_IPS, BLK * 4, 8, W),
                        axis=1).reshape(_IPS * 8, W)
    rs = xb.reshape(_IPS * (H // BLK), BLK, W).sum(axis=1)
    wi = lax.broadcasted_iota(jnp.int32, (W, W // BLK), 0)
    ci = lax.broadcasted_iota(jnp.int32, (W, W // BLK), 1)
    P = (wi // BLK == ci).astype(jnp.float32)
    bs_ref[0] = jnp.dot(rs, P, preferred_element_type=jnp.float32)


def _run_stats(xf):
    ng = BC // _IPS
    return pl.pallas_call(
        _stats_body,
        grid=(ng,),
        in_specs=[pl.BlockSpec((_IPS, H, W), lambda i: (i, 0, 0))],
        out_specs=[
            pl.BlockSpec((1, _IPS * 8, W), lambda i: (i, 0, 0)),
            pl.BlockSpec((1, _IPS * 8, W), lambda i: (i, 0, 0)),
            pl.BlockSpec((1, _IPS * (H // BLK), W // BLK),
                         lambda i: (i, 0, 0)),
        ],
        out_shape=[
            jax.ShapeDtypeStruct((ng, _IPS * 8, W), jnp.float32),
            jax.ShapeDtypeStruct((ng, _IPS * 8, W), jnp.float32),
            jax.ShapeDtypeStruct((ng, _IPS * (H // BLK), W // BLK),
                                 jnp.float32),
        ],
        compiler_params=pltpu.CompilerParams(
            dimension_semantics=("arbitrary",)),
    )(xf)


def _lut_body(bs_ref, mn_ref, mx_ref, w1_ref, b1_ref, w2_ref, b2_ref,
              w3_ref, b3_ref, alpha_ref, lut_ref, lutd_ref, s_ref, t_ref):
    K = NUM_BINS
    G = GROUP
    xmn = jnp.min(mn_ref[...], axis=1, keepdims=True)
    xmx = jnp.max(mx_ref[...], axis=1, keepdims=True)
    rng = xmx - xmn
    inv = 1.0 / (rng + 1e-6)
    xs = (bs_ref[...] * (1.0 / (BLK * BLK)) - xmn) * inv
    ji = lax.broadcasted_iota(jnp.int32, (B * G, BC), 0)
    bci = lax.broadcasted_iota(jnp.int32, (B * G, BC), 1)
    bg = (bci // C) * G + (bci % C) // (C // G)
    gsel = jnp.where(bg == ji, 1.0 / (C // G), 0.0)
    xg = jnp.dot(gsel, xs, preferred_element_type=jnp.float32)
    idx = jnp.clip(jnp.round(xg * (K - 1)).astype(jnp.int32), 0, K - 1)
    ki = lax.broadcasted_iota(jnp.int32, (B * G, xg.shape[1], K), 2)
    oh = (idx[:, :, None] == ki).astype(jnp.float32)
    hist = oh.sum(axis=1)
    pdf = hist / (hist.sum(axis=-1, keepdims=True) + 1e-6)
    r0 = lax.broadcasted_iota(jnp.int32, (K, K), 0)
    r1 = lax.broadcasted_iota(jnp.int32, (K, K), 1)
    T = (r0 <= r1).astype(jnp.float32)
    cdf = jnp.dot(pdf, T, preferred_element_type=jnp.float32)
    dc = 0.5 * (cdf[:G] + cdf[G:])
    z2 = jnp.zeros((G, 2), jnp.float32)
    dpad = jnp.concatenate([z2, dc, z2], axis=1)
    col = jnp.concatenate([dpad[:, t:t + K] for t in range(5)], axis=0)
    h = jnp.maximum(
        jnp.dot(w1_ref[...], col, preferred_element_type=jnp.float32)
        + b1_ref[...], 0.0)
    z2h = jnp.zeros((HIDDEN, 2), jnp.float32)
    hpad = jnp.concatenate([z2h, h, z2h], axis=1)
    w2 = w2_ref[...]
    h2 = b2_ref[...]
    for t in range(5):
        h2 = h2 + w2[:, t:t + 1] * hpad[:, t:t + K]
    h2 = jnp.maximum(h2, 0.0)
    delta = (jnp.dot(w3_ref[...], h2, preferred_element_type=jnp.float32)
             + b3_ref[...])
    sp = jnp.maximum(delta, 0.0) + jnp.log(1.0 + jnp.exp(-jnp.abs(delta)))
    cdf2 = jnp.dot(sp, T, preferred_element_type=jnp.float32)
    cdf2 = cdf2 / (cdf2[:, K - 1:K] + 1e-6)
    ident = lax.broadcasted_iota(jnp.int32, (G, K), 1).astype(jnp.float32)
    ident = ident * (1.0 / (K - 1))
    a = 1.0 / (1.0 + jnp.exp(-jnp.full((G, K), alpha_ref[0, 0])))
    lut2 = a * (cdf2 + ident) + (1.0 - a) * ident
    lutc = jnp.broadcast_to(lut2[:, None, :], (G, C // G, K)).reshape(C, K)
    lutbc = jnp.broadcast_to(lutc[None], (B, C, K)).reshape(BC, K)
    lut3 = lutbc * rng + xmn
    lut_ref[...] = lut3
    lutd_ref[...] = jnp.concatenate(
        [lut3[:, 1:] - lut3[:, :-1], jnp.zeros((BC, 1), jnp.float32)], axis=1)
    s = (K - 1.0) * inv
    s_ref[...] = jnp.broadcast_to(s, (BC, _L))
    t_ref[...] = jnp.broadcast_to(-(xmn * s), (BC, _L))


def _run_lut(bs2, xmn, xmx, w1e, b1c, w2e, b2c, w3r, b3c, alpha2):
    n_in = 9
    return pl.pallas_call(
        _lut_body,
        in_specs=[pl.BlockSpec(memory_space=pltpu.VMEM)] * n_in
        + [pl.BlockSpec(memory_space=pltpu.SMEM)],
        out_specs=[pl.BlockSpec(memory_space=pltpu.VMEM)] * 4,
        out_shape=[
            jax.ShapeDtypeStruct((BC, NUM_BINS), jnp.float32),
            jax.ShapeDtypeStruct((BC, NUM_BINS), jnp.float32),
            jax.ShapeDtypeStruct((BC, _L), jnp.float32),
            jax.ShapeDtypeStruct((BC, _L), jnp.float32),
        ],
    )(bs2, xmn, xmx, w1e, b1c, w2e, b2c, w3r, b3c, alpha2)


def _pix_body(x_hbm, lut_hbm, lutd_hbm, s_hbm, t_hbm, out_hbm,
              lut_v, lutd_v, s_v, t_v, b0, b1, b2,
              si0, si1, si2, so0, so1, so2):
    wid = lax.axis_index("s") * _NC + lax.axis_index("c")
    cbase = wid * CPW
    pltpu.sync_copy(lut_hbm.at[pl.ds(cbase * NUM_BINS, CPW * NUM_BINS)], lut_v)
    pltpu.sync_copy(lutd_hbm.at[pl.ds(cbase * NUM_BINS, CPW * NUM_BINS)],
                    lutd_v)
    pltpu.sync_copy(s_hbm.at[pl.ds(cbase * _L, CPW * _L)], s_v)
    pltpu.sync_copy(t_hbm.at[pl.ds(cbase * _L, CPW * _L)], t_v)
    nch = CPW * NCHUNK
    bufs, sins, souts = (b0, b1, b2), (si0, si1, si2), (so0, so1, so2)

    def in_sl(ch):
        return x_hbm.at[cbase + ch // NCHUNK, ch % NCHUNK, :, :]

    def out_sl(ch):
        return out_hbm.at[cbase + ch // NCHUNK, ch % NCHUNK, :, :]

    pltpu.async_copy(in_sl(0), b0, si0)
    pltpu.async_copy(in_sl(1), b1, si1)

    def group(g, carry):
        for b in range(3):
            ch = 3 * g + b
            buf, si, so = bufs[b], sins[b], souts[b]
            nb = (b + 2) % 3
            pltpu.make_async_copy(in_sl(ch), buf, si).wait()
            cl = ch // NCHUNK
            sv = s_v[pl.ds(cl * _L, _L)]
            tv = t_v[pl.ds(cl * _L, _L)]
            base_vec = jnp.full((_L,), cl * NUM_BINS, jnp.int32)

            @plsc.parallel_loop(0, CHUNK, _L, unroll=8)
            def pix(off, buf=buf, sv=sv, tv=tv, base_vec=base_vec):
                r = off // W
                c = off % W
                xv = buf[r, pl.ds(c, _L)]
                pos = xv * sv + tv
                idl = pos.astype(jnp.int32)
                frac = pos - idl.astype(jnp.float32)
                fl = base_vec + idl
                lo = plsc.load_gather(lut_v, [fl])
                dd = plsc.load_gather(lutd_v, [fl])
                buf[r, pl.ds(c, _L)] = lo + frac * dd

            pltpu.async_copy(buf, out_sl(ch), so)

            @pl.when(ch + 2 < nch)
            def _():
                @pl.when(ch >= 1)
                def _():
                    pltpu.make_async_copy(
                        bufs[nb], out_sl(ch - 1), souts[nb]).wait()
                pltpu.async_copy(in_sl(ch + 2), bufs[nb], sins[nb])
        return carry

    lax.fori_loop(0, nch // 3, group, 0)
    for j in range(3):
        pltpu.make_async_copy(bufs[j], out_sl(nch - 3 + j), souts[j]).wait()


_CROWS = CHUNK // W

_pix_kernel = functools.partial(
    pl.kernel,
    out_type=jax.ShapeDtypeStruct((BC, NCHUNK, _CROWS, W), jnp.float32),
    mesh=plsc.VectorSubcoreMesh(
        core_axis_name="c", subcore_axis_name="s",
        num_cores=_NC, num_subcores=_NS),
    compiler_params=pltpu.CompilerParams(
        needs_layout_passes=False, use_tc_tiling_on_sc=True),
    scratch_types=[
        pltpu.VMEM((CPW * NUM_BINS,), jnp.float32),
        pltpu.VMEM((CPW * NUM_BINS,), jnp.float32),
        pltpu.VMEM((CPW * _L,), jnp.float32),
        pltpu.VMEM((CPW * _L,), jnp.float32),
        pltpu.VMEM((_CROWS, W), jnp.float32),
        pltpu.VMEM((_CROWS, W), jnp.float32),
        pltpu.VMEM((_CROWS, W), jnp.float32),
        pltpu.SemaphoreType.DMA,
        pltpu.SemaphoreType.DMA,
        pltpu.SemaphoreType.DMA,
        pltpu.SemaphoreType.DMA,
        pltpu.SemaphoreType.DMA,
        pltpu.SemaphoreType.DMA,
    ],
)(_pix_body)


@jax.jit
def kernel(x, W1, b1, W2, b2, W3, b3, alpha):
    xf = x.reshape(BC, H, W)
    mn, mx, bs = _run_stats(xf)
    xmn = mn.reshape(BC, 8 * W)
    xmx = mx.reshape(BC, 8 * W)
    bs2 = bs.reshape(BC, (H // BLK) * (W // BLK))
    w1e = W1[:, :, 2, :].transpose(0, 2, 1).reshape(HIDDEN, 5 * GROUP)
    w2e = W2[:, 0, 2, :]
    w3r = W3[:, :, 0, 0]
    lut3, lutd, s_rep, t_rep = _run_lut(
        bs2, xmn, xmx, w1e, b1.reshape(HIDDEN, 1), w2e,
        b2.reshape(HIDDEN, 1), w3r, b3.reshape(GROUP, 1),
        alpha.reshape(1, 1))
    out = _pix_kernel(x.reshape(BC, NCHUNK, H // NCHUNK, W), lut3.reshape(-1),
                      lutd.reshape(-1), s_rep.reshape(-1), t_rep.reshape(-1))
    return out.reshape(B, C, H, W)


# speedup vs baseline: 1.3004x; 1.0441x over previous
import functools

import jax
import jax.numpy as jnp
from jax import lax
from jax.experimental import pallas as pl
from jax.experimental.pallas import tpu as pltpu
from jax.experimental.pallas import tpu_sc as plsc

NUM_BINS = 64
GROUP = 16
HIDDEN = 128

B, C, H, W = 2, 96, 512, 512
BC = B * C
NPIX = H * W
BLK = 16

_NC, _NS, _L = 2, 16, 16
_NW = _NC * _NS
CPW = BC // _NW
CHUNK = 32768
NCHUNK = NPIX // CHUNK


_IPS = 8


def _stats_body(x_ref, mn_ref, mx_ref, bs_ref):
    xb = x_ref[...]
    mn_ref[0] = jnp.min(xb.reshape(_IPS, BLK * 4, 8, W),
                        axis=1).reshape(_IPS * 8, W)
    mx_ref[0] = jnp.max(xb.reshape(_IPS, BLK * 4, 8, W),
                        axis=1).reshape(_IPS * 8, W)
    rs = xb.reshape(_IPS * (H // BLK), BLK, W).sum(axis=1)
    wi = lax.broadcasted_iota(jnp.int32, (W, W // BLK), 0)
    ci = lax.broadcasted_iota(jnp.int32, (W, W // BLK), 1)
    P = (wi // BLK == ci).astype(jnp.float32)
    bs_ref[0] = jnp.dot(rs, P, preferred_element_type=jnp.float32)


def _run_stats(xf):
    ng = BC // _IPS
    return pl.pallas_call(
        _stats_body,
        grid=(ng,),
        in_specs=[pl.BlockSpec((_IPS, H, W), lambda i: (i, 0, 0))],
        out_specs=[
            pl.BlockSpec((1, _IPS * 8, W), lambda i: (i, 0, 0)),
            pl.BlockSpec((1, _IPS * 8, W), lambda i: (i, 0, 0)),
            pl.BlockSpec((1, _IPS * (H // BLK), W // BLK),
                         lambda i: (i, 0, 0)),
        ],
        out_shape=[
            jax.ShapeDtypeStruct((ng, _IPS * 8, W), jnp.float32),
            jax.ShapeDtypeStruct((ng, _IPS * 8, W), jnp.float32),
            jax.ShapeDtypeStruct((ng, _IPS * (H // BLK), W // BLK),
                                 jnp.float32),
        ],
        compiler_params=pltpu.CompilerParams(
            dimension_semantics=("arbitrary",)),
    )(xf)


def _lut_body(bs_ref, mn_ref, mx_ref, w1_ref, b1_ref, w2_ref, b2_ref,
              w3_ref, b3_ref, alpha_ref, lut_ref, lutd_ref, s_ref, t_ref):
    K = NUM_BINS
    G = GROUP
    xmn = jnp.min(mn_ref[...], axis=1, keepdims=True)
    xmx = jnp.max(mx_ref[...], axis=1, keepdims=True)
    rng = xmx - xmn
    inv = 1.0 / (rng + 1e-6)
    xs = (bs_ref[...] * (1.0 / (BLK * BLK)) - xmn) * inv
    ji = lax.broadcasted_iota(jnp.int32, (B * G, BC), 0)
    bci = lax.broadcasted_iota(jnp.int32, (B * G, BC), 1)
    bg = (bci // C) * G + (bci % C) // (C // G)
    gsel = jnp.where(bg == ji, 1.0 / (C // G), 0.0)
    xg = jnp.dot(gsel, xs, preferred_element_type=jnp.float32)
    idx = jnp.clip(jnp.round(xg * (K - 1)).astype(jnp.int32), 0, K - 1)
    ki = lax.broadcasted_iota(jnp.int32, (B * G, xg.shape[1], K), 2)
    oh = (idx[:, :, None] == ki).astype(jnp.float32)
    hist = oh.sum(axis=1)
    pdf = hist / (hist.sum(axis=-1, keepdims=True) + 1e-6)
    r0 = lax.broadcasted_iota(jnp.int32, (K, K), 0)
    r1 = lax.broadcasted_iota(jnp.int32, (K, K), 1)
    T = (r0 <= r1).astype(jnp.float32)
    cdf = jnp.dot(pdf, T, preferred_element_type=jnp.float32)
    dc = 0.5 * (cdf[:G] + cdf[G:])
    z2 = jnp.zeros((G, 2), jnp.float32)
    dpad = jnp.concatenate([z2, dc, z2], axis=1)
    col = jnp.concatenate([dpad[:, t:t + K] for t in range(5)], axis=0)
    h = jnp.maximum(
        jnp.dot(w1_ref[...], col, preferred_element_type=jnp.float32)
        + b1_ref[...], 0.0)
    z2h = jnp.zeros((HIDDEN, 2), jnp.float32)
    hpad = jnp.concatenate([z2h, h, z2h], axis=1)
    w2 = w2_ref[...]
    h2 = b2_ref[...]
    for t in range(5):
        h2 = h2 + w2[:, t:t + 1] * hpad[:, t:t + K]
    h2 = jnp.maximum(h2, 0.0)
    delta = (jnp.dot(w3_ref[...], h2, preferred_element_type=jnp.float32)
             + b3_ref[...])
    sp = jnp.maximum(delta, 0.0) + jnp.log(1.0 + jnp.exp(-jnp.abs(delta)))
    cdf2 = jnp.dot(sp, T, preferred_element_type=jnp.float32)
    cdf2 = cdf2 / (cdf2[:, K - 1:K] + 1e-6)
    ident = lax.broadcasted_iota(jnp.int32, (G, K), 1).astype(jnp.float32)
    ident = ident * (1.0 / (K - 1))
    a = 1.0 / (1.0 + jnp.exp(-jnp.full((G, K), alpha_ref[0, 0])))
    lut2 = a * (cdf2 + ident) + (1.0 - a) * ident
    lutc = jnp.broadcast_to(lut2[:, None, :], (G, C // G, K)).reshape(C, K)
    lutbc = jnp.broadcast_to(lutc[None], (B, C, K)).reshape(BC, K)
    lut3 = lutbc * rng + xmn
    lut_ref[...] = lut3
    lutd_ref[...] = jnp.concatenate(
        [lut3[:, 1:] - lut3[:, :-1], jnp.zeros((BC, 1), jnp.float32)], axis=1)
    s = (K - 1.0) * inv
    s_ref[...] = jnp.broadcast_to(s, (BC, _L))
    t_ref[...] = jnp.broadcast_to(-(xmn * s), (BC, _L))


def _run_lut(bs2, xmn, xmx, w1e, b1c, w2e, b2c, w3r, b3c, alpha2):
    n_in = 9
    return pl.pallas_call(
        _lut_body,
        in_specs=[pl.BlockSpec(memory_space=pltpu.VMEM)] * n_in
        + [pl.BlockSpec(memory_space=pltpu.SMEM)],
        out_specs=[pl.BlockSpec(memory_space=pltpu.VMEM)] * 4,
        out_shape=[
            jax.ShapeDtypeStruct((BC, NUM_BINS), jnp.float32),
            jax.ShapeDtypeStruct((BC, NUM_BINS), jnp.float32),
            jax.ShapeDtypeStruct((BC, _L), jnp.float32),
            jax.ShapeDtypeStruct((BC, _L), jnp.float32),
        ],
    )(bs2, xmn, xmx, w1e, b1c, w2e, b2c, w3r, b3c, alpha2)


def _pix_body(x_hbm, lut_hbm, lutd_hbm, s_hbm, t_hbm, out_hbm,
              lut_v, lutd_v, s_v, t_v, b0, b1, b2,
              si0, si1, si2, so0, so1, so2):
    wid = lax.axis_index("s") * _NC + lax.axis_index("c")
    cbase = wid * CPW
    pltpu.sync_copy(lut_hbm.at[pl.ds(cbase * NUM_BINS, CPW * NUM_BINS)], lut_v)
    pltpu.sync_copy(lutd_hbm.at[pl.ds(cbase * NUM_BINS, CPW * NUM_BINS)],
                    lutd_v)
    pltpu.sync_copy(s_hbm.at[pl.ds(cbase * _L, CPW * _L)], s_v)
    pltpu.sync_copy(t_hbm.at[pl.ds(cbase * _L, CPW * _L)], t_v)
    nch = CPW * NCHUNK
    bufs, sins, souts = (b0, b1, b2), (si0, si1, si2), (so0, so1, so2)

    def in_sl(ch):
        return x_hbm.at[cbase + ch // NCHUNK, ch % NCHUNK, :, :]

    def out_sl(ch):
        return out_hbm.at[cbase + ch // NCHUNK, ch % NCHUNK, :, :]

    pltpu.async_copy(in_sl(0), b0, si0)
    pltpu.async_copy(in_sl(1), b1, si1)

    def group(g, carry):
        for b in range(3):
            ch = 3 * g + b
            buf, si, so = bufs[b], sins[b], souts[b]
            nb = (b + 2) % 3
            pltpu.make_async_copy(in_sl(ch), buf, si).wait()
            cl = ch // NCHUNK
            sv = s_v[pl.ds(cl * _L, _L)]
            tv = t_v[pl.ds(cl * _L, _L)]
            base_vec = jnp.full((_L,), cl * NUM_BINS, jnp.int32)

            @plsc.parallel_loop(0, CHUNK, _L, unroll=8)
            def pix(off, buf=buf, sv=sv, tv=tv, base_vec=base_vec):
                r = off // W
                c = off % W
                xv = buf[r, pl.ds(c, _L)]
                pos = xv * sv + tv
                idl = pos.astype(jnp.int32)
                frac = pos - idl.astype(jnp.float32)
                fl = base_vec + idl
                lo = plsc.load_gather(lut_v, [fl])
                dd = plsc.load_gather(lutd_v, [fl])
                buf[r, pl.ds(c, _L)] = lo + frac * dd

            pltpu.async_copy(buf, out_sl(ch), so)

            @pl.when(ch + 2 < nch)
            def _():
                @pl.when(ch >= 1)
                def _():
                    pltpu.make_async_copy(
                        bufs[nb], out_sl(ch - 1), souts[nb]).wait()
                pltpu.async_copy(in_sl(ch + 2), bufs[nb], sins[nb])
        return carry

    lax.fori_loop(0, nch // 3, group, 0)
    for j in range(3):
        pltpu.make_async_copy(bufs[j], out_sl(nch - 3 + j), souts[j]).wait()


_CROWS = CHUNK // W

_pix_kernel = functools.partial(
    pl.kernel,
    out_type=jax.ShapeDtypeStruct((BC, NCHUNK, _CROWS, W), jnp.float32),
    mesh=plsc.VectorSubcoreMesh(
        core_axis_name="c", subcore_axis_name="s",
        num_cores=_NC, num_subcores=_NS),
    compiler_params=pltpu.CompilerParams(
        needs_layout_passes=False, use_tc_tiling_on_sc=True),
    scratch_types=[
        pltpu.VMEM((CPW * NUM_BINS,), jnp.float32),
        pltpu.VMEM((CPW * NUM_BINS,), jnp.float32),
        pltpu.VMEM((CPW * _L,), jnp.float32),
        pltpu.VMEM((CPW * _L,), jnp.float32),
        pltpu.VMEM((_CROWS, W), jnp.float32),
        pltpu.VMEM((_CROWS, W), jnp.float32),
        pltpu.VMEM((_CROWS, W), jnp.float32),
        pltpu.SemaphoreType.DMA,
        pltpu.SemaphoreType.DMA,
        pltpu.SemaphoreType.DMA,
        pltpu.SemaphoreType.DMA,
        pltpu.SemaphoreType.DMA,
        pltpu.SemaphoreType.DMA,
    ],
)(_pix_body)


@jax.jit
def kernel(x, W1, b1, W2, b2, W3, b3, alpha):
    xf = x.reshape(BC, H, W)
    mn, mx, bs = _run_stats(xf)
    xmn = mn.reshape(BC, 8 * W)
    xmx = mx.reshape(BC, 8 * W)
    bs2 = bs.reshape(BC, (H // BLK) * (W // BLK))
    w1e = W1[:, :, 2, :].transpose(0, 2, 1).reshape(HIDDEN, 5 * GROUP)
    w2e = W2[:, 0, 2, :]
    w3r = W3[:, :, 0, 0]
    lut3, lutd, s_rep, t_rep = _run_lut(
        bs2, xmn, xmx, w1e, b1.reshape(HIDDEN, 1), w2e,
        b2.reshape(HIDDEN, 1), w3r, b3.reshape(GROUP, 1),
        alpha.reshape(1, 1))
    out = _pix_kernel(x.reshape(BC, NCHUNK, H // NCHUNK, W), lut3.reshape(-1),
                      lutd.reshape(-1), s_rep.reshape(-1), t_rep.reshape(-1))
    return out.reshape(B, C, H, W)


# speedup vs baseline: 1.3007x; 1.0002x over previous
import functools

import jax
import jax.numpy as jnp
from jax import lax
from jax.experimental import pallas as pl
from jax.experimental.pallas import tpu as pltpu
from jax.experimental.pallas import tpu_sc as plsc

NUM_BINS = 64
GROUP = 16
HIDDEN = 128

B, C, H, W = 2, 96, 512, 512
BC = B * C
NPIX = H * W
BLK = 16

_NC, _NS, _L = 2, 16, 16
_NW = _NC * _NS
CPW = BC // _NW
CHUNK = 32768
NCHUNK = NPIX // CHUNK


_IPS = 8


def _stats_body(x_ref, mn_ref, mx_ref, bs_ref):
    xb = x_ref[...]
    mn_ref[0] = jnp.min(xb.reshape(_IPS, BLK * 4, 8, W),
                        axis=1).reshape(_IPS * 8, W)
    mx_ref[0] = jnp.max(xb.reshape(_IPS, BLK * 4, 8, W),
                        axis=1).reshape(_IPS * 8, W)
    rs = xb.reshape(_IPS * (H // BLK), BLK, W).sum(axis=1)
    wi = lax.broadcasted_iota(jnp.int32, (W, W // BLK), 0)
    ci = lax.broadcasted_iota(jnp.int32, (W, W // BLK), 1)
    P = (wi // BLK == ci).astype(jnp.float32)
    bs_ref[0] = jnp.dot(rs, P, preferred_element_type=jnp.float32)


def _run_stats(xf):
    ng = BC // _IPS
    return pl.pallas_call(
        _stats_body,
        grid=(ng,),
        in_specs=[pl.BlockSpec((_IPS, H, W), lambda i: (i, 0, 0))],
        out_specs=[
            pl.BlockSpec((1, _IPS * 8, W), lambda i: (i, 0, 0)),
            pl.BlockSpec((1, _IPS * 8, W), lambda i: (i, 0, 0)),
            pl.BlockSpec((1, _IPS * (H // BLK), W // BLK),
                         lambda i: (i, 0, 0)),
        ],
        out_shape=[
            jax.ShapeDtypeStruct((ng, _IPS * 8, W), jnp.float32),
            jax.ShapeDtypeStruct((ng, _IPS * 8, W), jnp.float32),
            jax.ShapeDtypeStruct((ng, _IPS * (H // BLK), W // BLK),
                                 jnp.float32),
        ],
        compiler_params=pltpu.CompilerParams(
            dimension_semantics=("arbitrary",)),
    )(xf)


def _lut_body(bs_ref, mn_ref, mx_ref, w1_ref, b1_ref, w2_ref, b2_ref,
              w3_ref, b3_ref, alpha_ref, lut_ref, lutd_ref, s_ref, t_ref):
    K = NUM_BINS
    G = GROUP
    xmn = jnp.min(mn_ref[...], axis=1, keepdims=True)
    xmx = jnp.max(mx_ref[...], axis=1, keepdims=True)
    rng = xmx - xmn
    inv = 1.0 / (rng + 1e-6)
    xs = (bs_ref[...] * (1.0 / (BLK * BLK)) - xmn) * inv
    ji = lax.broadcasted_iota(jnp.int32, (B * G, BC), 0)
    bci = lax.broadcasted_iota(jnp.int32, (B * G, BC), 1)
    bg = (bci // C) * G + (bci % C) // (C // G)
    gsel = jnp.where(bg == ji, 1.0 / (C // G), 0.0)
    xg = jnp.dot(gsel, xs, preferred_element_type=jnp.float32)
    idx = jnp.clip(jnp.round(xg * (K - 1)).astype(jnp.int32), 0, K - 1)
    ki = lax.broadcasted_iota(jnp.int32, (B * G, xg.shape[1], K), 2)
    oh = (idx[:, :, None] == ki).astype(jnp.float32)
    hist = oh.sum(axis=1)
    pdf = hist / (hist.sum(axis=-1, keepdims=True) + 1e-6)
    r0 = lax.broadcasted_iota(jnp.int32, (K, K), 0)
    r1 = lax.broadcasted_iota(jnp.int32, (K, K), 1)
    T = (r0 <= r1).astype(jnp.float32)
    cdf = jnp.dot(pdf, T, preferred_element_type=jnp.float32)
    dc = 0.5 * (cdf[:G] + cdf[G:])
    z2 = jnp.zeros((G, 2), jnp.float32)
    dpad = jnp.concatenate([z2, dc, z2], axis=1)
    col = jnp.concatenate([dpad[:, t:t + K] for t in range(5)], axis=0)
    h = jnp.maximum(
        jnp.dot(w1_ref[...], col, preferred_element_type=jnp.float32)
        + b1_ref[...], 0.0)
    z2h = jnp.zeros((HIDDEN, 2), jnp.float32)
    hpad = jnp.concatenate([z2h, h, z2h], axis=1)
    w2 = w2_ref[...]
    h2 = b2_ref[...]
    for t in range(5):
        h2 = h2 + w2[:, t:t + 1] * hpad[:, t:t + K]
    h2 = jnp.maximum(h2, 0.0)
    delta = (jnp.dot(w3_ref[...], h2, preferred_element_type=jnp.float32)
             + b3_ref[...])
    sp = jnp.maximum(delta, 0.0) + jnp.log(1.0 + jnp.exp(-jnp.abs(delta)))
    cdf2 = jnp.dot(sp, T, preferred_element_type=jnp.float32)
    cdf2 = cdf2 / (cdf2[:, K - 1:K] + 1e-6)
    ident = lax.broadcasted_iota(jnp.int32, (G, K), 1).astype(jnp.float32)
    ident = ident * (1.0 / (K - 1))
    a = 1.0 / (1.0 + jnp.exp(-jnp.full((G, K), alpha_ref[0, 0])))
    lut2 = a * (cdf2 + ident) + (1.0 - a) * ident
    lutc = jnp.broadcast_to(lut2[:, None, :], (G, C // G, K)).reshape(C, K)
    lutbc = jnp.broadcast_to(lutc[None], (B, C, K)).reshape(BC, K)
    lut3 = lutbc * rng + xmn
    lut_ref[...] = lut3
    lutd_ref[...] = jnp.concatenate(
        [lut3[:, 1:] - lut3[:, :-1], jnp.zeros((BC, 1), jnp.float32)], axis=1)
    s = (K - 1.0) * inv
    s_ref[...] = jnp.broadcast_to(s, (BC, _L))
    t_ref[...] = jnp.broadcast_to(-(xmn * s), (BC, _L))


def _run_lut(bs2, xmn, xmx, w1e, b1c, w2e, b2c, w3r, b3c, alpha2):
    n_in = 9
    return pl.pallas_call(
        _lut_body,
        in_specs=[pl.BlockSpec(memory_space=pltpu.VMEM)] * n_in
        + [pl.BlockSpec(memory_space=pltpu.SMEM)],
        out_specs=[pl.BlockSpec(memory_space=pltpu.VMEM)] * 4,
        out_shape=[
            jax.ShapeDtypeStruct((BC, NUM_BINS), jnp.float32),
            jax.ShapeDtypeStruct((BC, NUM_BINS), jnp.float32),
            jax.ShapeDtypeStruct((BC, _L), jnp.float32),
            jax.ShapeDtypeStruct((BC, _L), jnp.float32),
        ],
    )(bs2, xmn, xmx, w1e, b1c, w2e, b2c, w3r, b3c, alpha2)


def _pix_body(x_hbm, lut_hbm, lutd_hbm, s_hbm, t_hbm, out_hbm,
              lut_v, lutd_v, s_v, t_v, b0, b1, b2,
              si0, si1, si2, so0, so1, so2):
    wid = lax.axis_index("s") * _NC + lax.axis_index("c")
    cbase = wid * CPW
    pltpu.sync_copy(lut_hbm.at[pl.ds(cbase * NUM_BINS, CPW * NUM_BINS)], lut_v)
    pltpu.sync_copy(lutd_hbm.at[pl.ds(cbase * NUM_BINS, CPW * NUM_BINS)],
                    lutd_v)
    pltpu.sync_copy(s_hbm.at[pl.ds(cbase * _L, CPW * _L)], s_v)
    pltpu.sync_copy(t_hbm.at[pl.ds(cbase * _L, CPW * _L)], t_v)
    nch = CPW * NCHUNK
    bufs, sins, souts = (b0, b1, b2), (si0, si1, si2), (so0, so1, so2)

    def in_sl(ch):
        return x_hbm.at[cbase + ch // NCHUNK, ch % NCHUNK, :, :]

    def out_sl(ch):
        return out_hbm.at[cbase + ch // NCHUNK, ch % NCHUNK, :, :]

    pltpu.async_copy(in_sl(0), b0, si0)
    pltpu.async_copy(in_sl(1), b1, si1)

    def group(g, carry):
        for b in range(3):
            ch = 3 * g + b
            buf, si, so = bufs[b], sins[b], souts[b]
            nb = (b + 2) % 3
            pltpu.make_async_copy(in_sl(ch), buf, si).wait()
            cl = ch // NCHUNK
            sv = s_v[pl.ds(cl * _L, _L)]
            tv = t_v[pl.ds(cl * _L, _L)]
            lutc = lut_v.at[pl.ds(cl * NUM_BINS, NUM_BINS)]
            lutdc = lutd_v.at[pl.ds(cl * NUM_BINS, NUM_BINS)]

            @plsc.parallel_loop(0, CHUNK, _L, unroll=8)
            def pix(off, buf=buf, sv=sv, tv=tv, lutc=lutc, lutdc=lutdc):
                r = off // W
                c = off % W
                xv = buf[r, pl.ds(c, _L)]
                pos = xv * sv + tv
                idl = pos.astype(jnp.int32)
                frac = pos - idl.astype(jnp.float32)
                lo = plsc.load_gather(lutc, [idl])
                dd = plsc.load_gather(lutdc, [idl])
                buf[r, pl.ds(c, _L)] = lo + frac * dd

            pltpu.async_copy(buf, out_sl(ch), so)

            @pl.when(ch + 2 < nch)
            def _():
                @pl.when(ch >= 1)
                def _():
                    pltpu.make_async_copy(
                        bufs[nb], out_sl(ch - 1), souts[nb]).wait()
                pltpu.async_copy(in_sl(ch + 2), bufs[nb], sins[nb])
        return carry

    lax.fori_loop(0, nch // 3, group, 0)
    for j in range(3):
        pltpu.make_async_copy(bufs[j], out_sl(nch - 3 + j), souts[j]).wait()


_CROWS = CHUNK // W

_pix_kernel = functools.partial(
    pl.kernel,
    out_type=jax.ShapeDtypeStruct((BC, NCHUNK, _CROWS, W), jnp.float32),
    mesh=plsc.VectorSubcoreMesh(
        core_axis_name="c", subcore_axis_name="s",
        num_cores=_NC, num_subcores=_NS),
    compiler_params=pltpu.CompilerParams(
        needs_layout_passes=False, use_tc_tiling_on_sc=True),
    scratch_types=[
        pltpu.VMEM((CPW * NUM_BINS,), jnp.float32),
        pltpu.VMEM((CPW * NUM_BINS,), jnp.float32),
        pltpu.VMEM((CPW * _L,), jnp.float32),
        pltpu.VMEM((CPW * _L,), jnp.float32),
        pltpu.VMEM((_CROWS, W), jnp.float32),
        pltpu.VMEM((_CROWS, W), jnp.float32),
        pltpu.VMEM((_CROWS, W), jnp.float32),
        pltpu.SemaphoreType.DMA,
        pltpu.SemaphoreType.DMA,
        pltpu.SemaphoreType.DMA,
        pltpu.SemaphoreType.DMA,
        pltpu.SemaphoreType.DMA,
        pltpu.SemaphoreType.DMA,
    ],
)(_pix_body)


@jax.jit
def kernel(x, W1, b1, W2, b2, W3, b3, alpha):
    xf = x.reshape(BC, H, W)
    mn, mx, bs = _run_stats(xf)
    xmn = mn.reshape(BC, 8 * W)
    xmx = mx.reshape(BC, 8 * W)
    bs2 = bs.reshape(BC, (H // BLK) * (W // BLK))
    w1e = W1[:, :, 2, :].transpose(0, 2, 1).reshape(HIDDEN, 5 * GROUP)
    w2e = W2[:, 0, 2, :]
    w3r = W3[:, :, 0, 0]
    lut3, lutd, s_rep, t_rep = _run_lut(
        bs2, xmn, xmx, w1e, b1.reshape(HIDDEN, 1), w2e,
        b2.reshape(HIDDEN, 1), w3r, b3.reshape(GROUP, 1),
        alpha.reshape(1, 1))
    out = _pix_kernel(x.reshape(BC, NCHUNK, H // NCHUNK, W), lut3.reshape(-1),
                      lutd.reshape(-1), s_rep.reshape(-1), t_rep.reshape(-1))
    return out.reshape(B, C, H, W)
